# Initial kernel scaffold; baseline (speedup 1.0000x reference)
import jax
import jax.numpy as jnp
from jax.experimental import pallas as pl


def kernel(user_feat, hi, edge_index, att_W1, att_b1, att_w2, att_b2, W, b):
    raise NotImplementedError("write your pallas kernel here")


# speedup vs baseline: 2.2693x; 2.2693x over previous
import functools

import jax
import jax.numpy as jnp
from jax.experimental import pallas as pl


D = 128


def _linear_body(agg_ref, w_ref, b_ref, out_ref):
    acc = jax.lax.dot_general(
        agg_ref[...], w_ref[...],
        (((1,), (1,)), ((), ())),
        preferred_element_type=jnp.float32,
    )
    out_ref[...] = acc + b_ref[...][None, :]


def kernel(user_feat, hi, edge_index, att_W1, att_b1, att_w2, att_b2, W, b):
    row = edge_index[0].astype(jnp.int32)
    col = edge_index[1].astype(jnp.int32)
    n = user_feat.shape[0]

    P = user_feat @ att_W1[:D] + att_b1
    Q = hi @ att_W1[D:]

    t = jnp.tanh(P[row] + Q[col])
    logits = t @ att_w2 + att_b2
    gmax = jnp.max(logits)
    ex = jnp.exp(logits - gmax)
    den = jax.ops.segment_sum(ex, col, num_segments=n)
    u = jax.ops.segment_sum(ex[:, None] * hi[row], col, num_segments=n)
    agg = u / jnp.where(den == 0.0, 1.0, den)[:, None]

    hs = pl.pallas_call(
        _linear_body,
        out_shape=jax.ShapeDtypeStruct((n, D), jnp.float32),
    )(agg, W, b)
    return hs


# speedup vs baseline: 2.3971x; 1.0563x over previous
import functools

import jax
import jax.numpy as jnp
from jax import lax
from jax.experimental import pallas as pl
from jax.experimental.pallas import tpu as pltpu
from jax.experimental.pallas import tpu_sc as plsc

N = 10000
E = 320000
D = 128

NC = 2
NS = 16
NW = NC * NS
EPW = E // NW
C = 16
NCHUNK = EPW // C
STRIPE = 640
DN = 1280
DSTRIPE = DN // NS


def _prep_body(uf_ref, hi_ref, w1_ref, b1_ref, w2_ref, b2_ref, col2d_ref,
               a_ref, q_ref, wm_ref, ws_ref, col8_ref):
    col8_ref[...] = lax.shift_right_logical(col2d_ref[...], 3)
    w1t = w1_ref[0:D, :]
    w1b = w1_ref[D:2 * D, :]
    p = lax.dot_general(uf_ref[...], w1t, (((1,), (0,)), ((), ())),
                        preferred_element_type=jnp.float32)
    a_ref[:, 0:D] = 2.0 * (p + b1_ref[...])
    a_ref[:, D:2 * D] = hi_ref[...]
    q = lax.dot_general(hi_ref[...], w1b, (((1,), (0,)), ((), ())),
                        preferred_element_type=jnp.float32)
    q_ref[...] = 2.0 * q
    w2 = w2_ref[...]
    wm_ref[...] = -2.0 * w2
    b2 = b2_ref[0, 0]
    shift = jnp.sum(jnp.abs(w2)) + jnp.abs(b2)
    sfold = b2 - shift + jnp.sum(w2)
    ws_ref[...] = jnp.full((1, 16), sfold, dtype=jnp.float32)


def _edge_body(a_hbm, q_hbm, row_hbm, col_hbm, col8_hbm, wc_hbm,
               u_out, d_out,
               bufA, bufQ, msgb, den, rowb, colb, col8b, wbuf, idx16,
               u_sh, d_sh, semA, semQ, semR, semC, sem8):
    cid = lax.axis_index("c")
    sid = lax.axis_index("s")
    wid = sid * NC + cid

    pltpu.sync_copy(wc_hbm, wbuf)

    zero16 = jnp.zeros((16,), jnp.float32)
    iota16 = lax.iota(jnp.int32, 16)
    zidx = jnp.zeros((16,), jnp.int32)

    def zrow(i, _):
        for j in range(8):
            bufQ[i, pl.ds(j * 16, 16)] = zero16
            den[i, pl.ds(j * 16, 16)] = zero16
        return 0
    lax.fori_loop(0, C, zrow, 0)

    ngroups = jnp.where(sid < NS - 1, STRIPE // 16, 400 // 16)
    sbase = sid * STRIPE

    def zgroup(i, _):
        idx16[...] = sbase + i * 16 + iota16
        pltpu.sync_copy(bufQ.at[pl.ds(0, 16)], u_sh.at[idx16])
        return 0
    lax.fori_loop(0, ngroups, zgroup, 0)

    dbase = sid * DSTRIPE

    def zgroup_d(i, _):
        idx16[...] = dbase + i * 16 + iota16
        pltpu.sync_copy(den.at[pl.ds(0, 16)], d_sh.at[idx16])
        return 0
    lax.fori_loop(0, DSTRIPE // 16, zgroup_d, 0)
    plsc.subcore_barrier()

    wvecs = [wbuf[pl.ds(j * 16, 16)] for j in range(8)]
    sfold_vec = wbuf[pl.ds(D, 16)]

    base = wid * EPW

    def chunk(ci, _):
        cb = base + ci * C
        cpR = pltpu.async_copy(row_hbm.at[pl.ds(cb, C)], rowb, semR)
        cpC = pltpu.async_copy(col_hbm.at[pl.ds(cb, C)], colb, semC)
        cp8 = pltpu.async_copy(col8_hbm.at[pl.ds(cb, C)], col8b, sem8)
        cpR.wait()
        cpC.wait()
        cp8.wait()
        cpA = pltpu.async_copy(a_hbm.at[rowb], bufA, semA)
        cpQ = pltpu.async_copy(q_hbm.at[colb], bufQ, semQ)
        cpA.wait()
        cpQ.wait()

        acc = sfold_vec
        for j in range(8):
            wj = wvecs[j]
            for l2 in range(16):
                dd = jnp.full((16,), j * 16 + l2, jnp.int32)
                cA = plsc.load_gather(bufA, [iota16, dd])
                cQ = plsc.load_gather(bufQ, [iota16, dd])
                acc = acc + wj[l2] / (jnp.exp(cA + cQ) + 1.0)
        exv = jnp.exp(acc)
        colv = colb[...]

        for l in range(16):
            ex_s = exv[l]
            cm = colv[l] & 7
            for k in range(8):
                msgb[l, pl.ds(k * 16, 16)] = \
                    ex_s * bufA[l, pl.ds(D + k * 16, 16)]
                den[l, pl.ds(k * 16, 16)] = jnp.where(
                    iota16 == 0, jnp.where(cm == k, ex_s, 0.0), 0.0)

        pltpu.sync_copy(msgb, u_sh.at[colv], add=True)
        pltpu.sync_copy(den, d_sh.at[col8b], add=True)
        return 0

    lax.fori_loop(0, NCHUNK, chunk, 0)
    plsc.subcore_barrier()

    def ogroup(i, _):
        idx16[...] = sbase + i * 16 + iota16
        pltpu.sync_copy(u_sh.at[idx16], bufQ.at[pl.ds(0, 16)])
        r0 = sbase + i * 16
        pltpu.sync_copy(bufQ.at[pl.ds(0, 16)],
                        u_out.at[pl.ds(cid * N + r0, 16)])
        return 0
    lax.fori_loop(0, ngroups, ogroup, 0)

    def ogroup_d(i, _):
        idx16[...] = dbase + i * 16 + iota16
        pltpu.sync_copy(d_sh.at[idx16], den.at[pl.ds(0, 16)])
        r0 = dbase + i * 16
        pltpu.sync_copy(den.at[pl.ds(0, 16)],
                        d_out.at[pl.ds(cid * DN + r0, 16)])
        return 0
    lax.fori_loop(0, DSTRIPE // 16, ogroup_d, 0)


_edge_pass = functools.partial(
    pl.kernel,
    out_type=[jax.ShapeDtypeStruct((NC * N, D), jnp.float32),
              jax.ShapeDtypeStruct((NC * DN, D), jnp.float32)],
    mesh=plsc.VectorSubcoreMesh(core_axis_name="c", subcore_axis_name="s"),
    compiler_params=pltpu.CompilerParams(needs_layout_passes=False),
    scratch_types=[
        pltpu.VMEM((C, 2 * D), jnp.float32),
        pltpu.VMEM((C, D), jnp.float32),
        pltpu.VMEM((C, D), jnp.float32),
        pltpu.VMEM((C, D), jnp.float32),
        pltpu.VMEM((C,), jnp.int32),
        pltpu.VMEM((C,), jnp.int32),
        pltpu.VMEM((C,), jnp.int32),
        pltpu.VMEM((D + 16,), jnp.float32),
        pltpu.VMEM((16,), jnp.int32),
        pltpu.VMEM_SHARED((N, D), jnp.float32),
        pltpu.VMEM_SHARED((DN, D), jnp.float32),
        pltpu.SemaphoreType.DMA,
        pltpu.SemaphoreType.DMA,
        pltpu.SemaphoreType.DMA,
        pltpu.SemaphoreType.DMA,
        pltpu.SemaphoreType.DMA,
    ],
)(_edge_body)


def _finish_body(u2_ref, dn_ref, w_ref, b_ref, out_ref):
    u = u2_ref[0:N, :] + u2_ref[N:2 * N, :]
    dn = dn_ref[...]
    dn = jnp.where(dn == 0.0, 1.0, dn)
    agg = u / dn
    acc = lax.dot_general(agg, w_ref[...], (((1,), (1,)), ((), ())),
                          preferred_element_type=jnp.float32)
    out_ref[...] = acc + b_ref[...]


def kernel(user_feat, hi, edge_index, att_W1, att_b1, att_w2, att_b2, W, b):
    row = edge_index[0].astype(jnp.int32)
    col = edge_index[1].astype(jnp.int32)

    A, Q, wm, ws, col8_2d = pl.pallas_call(
        _prep_body,
        out_shape=[
            jax.ShapeDtypeStruct((N, 2 * D), jnp.float32),
            jax.ShapeDtypeStruct((N, D), jnp.float32),
            jax.ShapeDtypeStruct((1, D), jnp.float32),
            jax.ShapeDtypeStruct((1, 16), jnp.float32),
            jax.ShapeDtypeStruct((E // D, D), jnp.int32),
        ],
    )(user_feat, hi, att_W1, att_b1.reshape(1, D), att_w2.reshape(1, D),
      att_b2.reshape(1, 1), col.reshape(E // D, D))

    wc = jnp.concatenate([wm[0], ws[0]])

    U2, D2 = _edge_pass(A, Q, row, col, col8_2d.reshape(E), wc)

    dnp = D2[:DN] + D2[DN:]
    dn = dnp[:, ::16].reshape(-1)[:N]

    hs = pl.pallas_call(
        _finish_body,
        out_shape=jax.ShapeDtypeStruct((N, D), jnp.float32),
    )(U2, dn.reshape(N, 1), W, b.reshape(1, D))
    return hs
